# Initial kernel scaffold; baseline (speedup 1.0000x reference)
import jax
import jax.numpy as jnp
from jax.experimental import pallas as pl


def kernel(ids, weight):
    raise NotImplementedError("write your pallas kernel here")


# speedup vs baseline: 5.1649x; 5.1649x over previous
import functools

import jax
import jax.numpy as jnp
from jax import lax
from jax.experimental import pallas as pl
from jax.experimental.pallas import tpu as pltpu
from jax.experimental.pallas import tpu_sc as plsc

NUM_ROWS = 100000
DIM = 128

NC = 2
NS = 16
NW = NC * NS

B_TOTAL = 4096 * 200
B_PER_W = B_TOTAL // NW
CHUNK = 128
N_CHUNKS = B_PER_W // CHUNK


def _body(ids_hbm, w_hbm, out_hbm, idx_v, rows_v, sem):
    wid = lax.axis_index("s") * NC + lax.axis_index("c")
    base = wid * B_PER_W

    def step(g, carry):
        off = base + g * CHUNK
        pltpu.sync_copy(ids_hbm.at[pl.ds(off, CHUNK)], idx_v)
        pltpu.async_copy(w_hbm.at[idx_v], rows_v, sem).wait()
        pltpu.sync_copy(rows_v, out_hbm.at[pl.ds(off, CHUNK)])
        return carry

    lax.fori_loop(0, N_CHUNKS, step, 0)


@functools.partial(jax.jit, static_argnames=())
def _run(ids_flat, weight):
    f = pl.kernel(
        _body,
        out_type=jax.ShapeDtypeStruct((B_TOTAL, DIM), jnp.float32),
        mesh=plsc.VectorSubcoreMesh(core_axis_name="c", subcore_axis_name="s"),
        scratch_types=[
            pltpu.VMEM((CHUNK,), jnp.int32),
            pltpu.VMEM((CHUNK, DIM), jnp.float32),
            pltpu.SemaphoreType.DMA,
        ],
    )
    return f(ids_flat, weight)


def kernel(ids, weight):
    ids_flat = ids.reshape(-1).astype(jnp.int32)
    out = _run(ids_flat, weight)
    return out.reshape(ids.shape[0], ids.shape[1], DIM)


# speedup vs baseline: 9.1790x; 1.7772x over previous
import functools

import jax
import jax.numpy as jnp
from jax import lax
from jax.experimental import pallas as pl
from jax.experimental.pallas import tpu as pltpu
from jax.experimental.pallas import tpu_sc as plsc

NUM_ROWS = 100000
DIM = 128

NC = 2
NS = 16
NW = NC * NS

B_TOTAL = 4096 * 200
B_PER_W = B_TOTAL // NW
CHUNK = 128
N_CHUNKS = B_PER_W // CHUNK
NBUF = 4
N_GROUPS = N_CHUNKS // NBUF


def _body(ids_hbm, w_hbm, out_hbm, idx2, rows, s0, s1, s2, s3):
    sems = (s0, s1, s2, s3)
    wid = lax.axis_index("s") * NC + lax.axis_index("c")
    base = wid * B_PER_W

    pltpu.sync_copy(ids_hbm.at[wid], idx2)

    def fire(c, b):
        pltpu.async_copy(w_hbm.at[idx2.at[c]], rows.at[b], sems[b])

    def wait(b):
        pltpu.make_async_copy(w_hbm.at[idx2.at[0]], rows.at[b], sems[b]).wait()

    for b in range(NBUF):
        fire(b, b)

    def outer(go, carry):
        for b in range(NBUF):
            g = go * NBUF + b
            wait(b)
            pltpu.sync_copy(rows.at[b], out_hbm.at[pl.ds(base + g * CHUNK, CHUNK)])
            fire(g + NBUF, b)
        return carry

    lax.fori_loop(0, N_GROUPS - 1, outer, 0)

    for b in range(NBUF):
        g = (N_GROUPS - 1) * NBUF + b
        wait(b)
        pltpu.sync_copy(rows.at[b], out_hbm.at[pl.ds(base + g * CHUNK, CHUNK)])


@jax.jit
def _run(ids3, weight):
    f = pl.kernel(
        _body,
        out_type=jax.ShapeDtypeStruct((B_TOTAL, DIM), jnp.float32),
        mesh=plsc.VectorSubcoreMesh(core_axis_name="c", subcore_axis_name="s"),
        scratch_types=[
            pltpu.VMEM((N_CHUNKS, CHUNK), jnp.int32),
            pltpu.VMEM((NBUF, CHUNK, DIM), jnp.float32),
            pltpu.SemaphoreType.DMA,
            pltpu.SemaphoreType.DMA,
            pltpu.SemaphoreType.DMA,
            pltpu.SemaphoreType.DMA,
        ],
    )
    return f(ids3, weight)


def kernel(ids, weight):
    ids3 = ids.reshape(NW, N_CHUNKS, CHUNK).astype(jnp.int32)
    out = _run(ids3, weight)
    return out.reshape(ids.shape[0], ids.shape[1], DIM)


# speedup vs baseline: 9.1937x; 1.0016x over previous
import functools

import jax
import jax.numpy as jnp
from jax import lax
from jax.experimental import pallas as pl
from jax.experimental.pallas import tpu as pltpu
from jax.experimental.pallas import tpu_sc as plsc

NUM_ROWS = 100000
DIM = 128

NC = 2
NS = 16
NW = NC * NS

B_TOTAL = 4096 * 200
B_PER_W = B_TOTAL // NW
CHUNK = 128
N_CHUNKS = B_PER_W // CHUNK
NBUF = 5
assert N_CHUNKS % NBUF == 0
N_GROUPS = N_CHUNKS // NBUF


def _body(ids_hbm, w_hbm, out_hbm, idx2, rows, s0, s1, s2, s3, s4):
    sems = (s0, s1, s2, s3, s4)
    wid = lax.axis_index("s") * NC + lax.axis_index("c")
    base = wid * B_PER_W

    pltpu.sync_copy(ids_hbm.at[wid], idx2)

    def fire(c, b):
        pltpu.async_copy(w_hbm.at[idx2.at[c]], rows.at[b], sems[b])

    def wait(b):
        pltpu.make_async_copy(w_hbm.at[idx2.at[0]], rows.at[b], sems[b]).wait()

    for b in range(NBUF):
        fire(b, b)

    def outer(go, carry):
        for b in range(NBUF):
            g = go * NBUF + b
            wait(b)
            pltpu.sync_copy(rows.at[b], out_hbm.at[pl.ds(base + g * CHUNK, CHUNK)])
            fire(g + NBUF, b)
        return carry

    lax.fori_loop(0, N_GROUPS - 1, outer, 0)

    for b in range(NBUF):
        g = (N_GROUPS - 1) * NBUF + b
        wait(b)
        pltpu.sync_copy(rows.at[b], out_hbm.at[pl.ds(base + g * CHUNK, CHUNK)])


@jax.jit
def _run(ids3, weight):
    f = pl.kernel(
        _body,
        out_type=jax.ShapeDtypeStruct((B_TOTAL, DIM), jnp.float32),
        mesh=plsc.VectorSubcoreMesh(core_axis_name="c", subcore_axis_name="s"),
        scratch_types=[
            pltpu.VMEM((N_CHUNKS, CHUNK), jnp.int32),
            pltpu.VMEM((NBUF, CHUNK, DIM), jnp.float32),
        ] + [pltpu.SemaphoreType.DMA] * NBUF,
    )
    return f(ids3, weight)


def kernel(ids, weight):
    ids3 = ids.reshape(NW, N_CHUNKS, CHUNK).astype(jnp.int32)
    out = _run(ids3, weight)
    return out.reshape(ids.shape[0], ids.shape[1], DIM)
